# Initial kernel scaffold; baseline (speedup 1.0000x reference)
import jax
import jax.numpy as jnp
from jax.experimental import pallas as pl


def kernel(x_user, x_movie, edge_label_index):
    raise NotImplementedError("write your pallas kernel here")


# speedup vs baseline: 3.2061x; 3.2061x over previous
import functools

import jax
import jax.numpy as jnp
from jax import lax
from jax.experimental import pallas as pl
from jax.experimental.pallas import tpu as pltpu
from jax.experimental.pallas import tpu_sc as plsc

NC = 2
NS = 16
L = 16
NW = NC * NS

B = 320000
D = 128
C = 80
BPW = B // NW
NCHUNK = BPW // C

_mesh = plsc.VectorSubcoreMesh(core_axis_name="c", subcore_axis_name="s")


@functools.partial(
    pl.kernel,
    mesh=_mesh,
    out_type=jax.ShapeDtypeStruct((B,), jnp.float32),
    compiler_params=pltpu.CompilerParams(needs_layout_passes=False),
    scratch_types=[
        pltpu.VMEM((C,), jnp.int32),
        pltpu.VMEM((C,), jnp.int32),
        pltpu.VMEM((C, D), jnp.float32),
        pltpu.VMEM((C, D), jnp.float32),
        pltpu.VMEM((C,), jnp.float32),
        pltpu.SemaphoreType.DMA,
        pltpu.SemaphoreType.DMA,
    ],
)
def _decode(xu_hbm, xm_hbm, ui_hbm, mi_hbm, out_hbm,
            ui_v, mi_v, u_v, m_v, o_v, sem_u, sem_m):
    wid = lax.axis_index("s") * NC + lax.axis_index("c")
    base = wid * BPW

    def chunk_body(i, carry):
        off = base + i * C
        pltpu.sync_copy(ui_hbm.at[pl.ds(off, C)], ui_v)
        pltpu.sync_copy(mi_hbm.at[pl.ds(off, C)], mi_v)
        cp_u = pltpu.async_copy(xu_hbm.at[ui_v], u_v, sem_u)
        cp_m = pltpu.async_copy(xm_hbm.at[mi_v], m_v, sem_m)
        cp_u.wait()
        cp_m.wait()

        lane0 = lax.iota(jnp.int32, L) == 0

        def edge_body(e, ecarry):
            acc = u_v[e, pl.ds(0, L)] * m_v[e, pl.ds(0, L)]
            for j in range(1, D // L):
                acc = acc + u_v[e, pl.ds(j * L, L)] * m_v[e, pl.ds(j * L, L)]
            s = jnp.broadcast_to(jnp.sum(acc), (L,))
            eidx = jnp.full((L,), e, jnp.int32)
            plsc.store_scatter(o_v, [eidx], s, mask=lane0)
            return ecarry

        lax.fori_loop(0, C, edge_body, 0)
        pltpu.sync_copy(o_v, out_hbm.at[pl.ds(off, C)])
        return carry

    lax.fori_loop(0, NCHUNK, chunk_body, 0)


def kernel(x_user, x_movie, edge_label_index):
    idx = edge_label_index.astype(jnp.int32)
    return _decode(x_user, x_movie, idx[0], idx[1])


# speedup vs baseline: 6.6852x; 2.0852x over previous
import functools

import jax
import jax.numpy as jnp
from jax import lax
from jax.experimental import pallas as pl
from jax.experimental.pallas import tpu as pltpu
from jax.experimental.pallas import tpu_sc as plsc

NC = 2
NS = 16
L = 16
NW = NC * NS

B = 320000
D = 128
C = 80
BPW = B // NW
NCHUNK = BPW // C

_mesh = plsc.VectorSubcoreMesh(core_axis_name="c", subcore_axis_name="s")


@functools.partial(
    pl.kernel,
    mesh=_mesh,
    out_type=jax.ShapeDtypeStruct((B,), jnp.float32),
    compiler_params=pltpu.CompilerParams(needs_layout_passes=False),
    scratch_types=[
        pltpu.VMEM((C,), jnp.int32),
        pltpu.VMEM((C,), jnp.int32),
        pltpu.VMEM((C,), jnp.int32),
        pltpu.VMEM((C,), jnp.int32),
        pltpu.VMEM((C, D), jnp.float32),
        pltpu.VMEM((C, D), jnp.float32),
        pltpu.VMEM((C, D), jnp.float32),
        pltpu.VMEM((C, D), jnp.float32),
        pltpu.VMEM((C,), jnp.float32),
        pltpu.VMEM((C,), jnp.float32),
        pltpu.SemaphoreType.DMA,
        pltpu.SemaphoreType.DMA,
        pltpu.SemaphoreType.DMA,
        pltpu.SemaphoreType.DMA,
        pltpu.SemaphoreType.DMA,
        pltpu.SemaphoreType.DMA,
    ],
)
def _decode(xu_hbm, xm_hbm, ui_hbm, mi_hbm, out_hbm,
            ui0, ui1, mi0, mi1, u0, u1, m0, m1, o0, o1,
            si0, si1, sg0, sg1, so0, so1):
    ui = (ui0, ui1)
    mi = (mi0, mi1)
    uv = (u0, u1)
    mv = (m0, m1)
    ov = (o0, o1)
    si = (si0, si1)
    sg = (sg0, sg1)
    so = (so0, so1)

    wid = lax.axis_index("s") * NC + lax.axis_index("c")
    base = wid * BPW
    lane0 = lax.iota(jnp.int32, L) == 0

    def fire_idx(c, b):
        off = base + c * C
        pltpu.async_copy(ui_hbm.at[pl.ds(off, C)], ui[b], si[b])
        pltpu.async_copy(mi_hbm.at[pl.ds(off, C)], mi[b], si[b])

    def wait_idx(b):
        pltpu.make_async_copy(ui_hbm.at[pl.ds(0, C)], ui[b], si[b]).wait()
        pltpu.make_async_copy(mi_hbm.at[pl.ds(0, C)], mi[b], si[b]).wait()

    def fire_gather(b):
        pltpu.async_copy(xu_hbm.at[ui[b]], uv[b], sg[b])
        pltpu.async_copy(xm_hbm.at[mi[b]], mv[b], sg[b])

    def wait_gather(b):
        pltpu.make_async_copy(xu_hbm.at[ui[b]], uv[b], sg[b]).wait()
        pltpu.make_async_copy(xm_hbm.at[mi[b]], mv[b], sg[b]).wait()

    def fire_out(c, b):
        off = base + c * C
        pltpu.async_copy(ov[b], out_hbm.at[pl.ds(off, C)], so[b])

    def wait_out(b):
        pltpu.make_async_copy(ov[b], out_hbm.at[pl.ds(0, C)], so[b]).wait()

    def compute(b):
        u_v, m_v, o_v = uv[b], mv[b], ov[b]

        def group_body(g, carry):
            e0 = g * L
            for k in range(L):
                e = e0 + k
                acc = u_v[e, pl.ds(0, L)] * m_v[e, pl.ds(0, L)]
                for j in range(1, D // L):
                    acc = acc + u_v[e, pl.ds(j * L, L)] * m_v[e, pl.ds(j * L, L)]
                s = jnp.broadcast_to(jnp.sum(acc), (L,))
                eidx = jnp.full((L,), e, jnp.int32)
                plsc.store_scatter(o_v, [eidx], s, mask=lane0)
            return carry

        lax.fori_loop(0, C // L, group_body, 0)

    fire_idx(0, 0)
    fire_idx(1, 1)
    wait_idx(0)
    fire_gather(0)

    def super_body(it, carry):
        for b in (0, 1):
            c = 2 * it + b
            nb = 1 - b
            wait_idx(nb)
            fire_gather(nb)
            wait_gather(b)
            fire_idx(jnp.minimum(c + 2, NCHUNK - 1), b)
            @pl.when(it > 0)
            def _():
                wait_out(b)
            compute(b)
            fire_out(c, b)
        return carry

    lax.fori_loop(0, (NCHUNK - 1) // 2, super_body, 0)

    wait_idx(1)
    wait_gather(0)
    wait_out(0)
    compute(0)
    fire_out(NCHUNK - 1, 0)
    wait_out(1)
    wait_out(0)


def kernel(x_user, x_movie, edge_label_index):
    idx = edge_label_index.astype(jnp.int32)
    return _decode(x_user, x_movie, idx[0], idx[1])
